# baseline (speedup 1.0000x reference)
import functools

import jax
import jax.numpy as jnp
from jax import lax
from jax.experimental import pallas as pl
from jax.experimental.pallas import tpu as pltpu
from jax.experimental.pallas import tpu_sc as plsc

NC, NS = 2, 16
NW = NC * NS
B, L, D = 1024, 200, 64
VECL = 16
BB = 128
NB = B // BB
L_PER_W = L // (NW // NB)

_mesh = plsc.VectorSubcoreMesh(core_axis_name="c", subcore_axis_name="s")


@functools.partial(
    pl.kernel,
    out_type=jax.ShapeDtypeStruct((L, D // 8, NB, 8, BB), jnp.float32),
    mesh=_mesh,
    compiler_params=pltpu.CompilerParams(use_tc_tiling_on_sc=False,
                                        needs_layout_passes=False),
    scratch_types=[
        pltpu.VMEM((L_PER_W, BB), jnp.int32),
        pltpu.VMEM((L_PER_W * D,), jnp.float32),
        pltpu.VMEM((2, BB, D), jnp.float32),
        pltpu.VMEM((2, D // 8, 8, BB), jnp.float32),
        pltpu.SemaphoreType.DMA((2,)),
        pltpu.SemaphoreType.DMA((2,)),
    ],
)
def _seq_embed(seq_t_hbm, tok_hbm, pos_hbm, out_hbm, idx_v, pos_v, gbuf, tbuf,
               gsem, ssem):
    wid = lax.axis_index("s") * NC + lax.axis_index("c")
    l0 = (wid // NB) * L_PER_W
    bhi = wid % NB

    pltpu.sync_copy(seq_t_hbm.at[pl.ds(l0, L_PER_W), pl.ds(bhi * BB, BB)], idx_v)
    pltpu.sync_copy(pos_hbm.at[pl.ds(l0 * D, L_PER_W * D)], pos_v)

    def gather_start(u, b):
        pltpu.async_copy(tok_hbm.at[idx_v.at[u]], gbuf.at[b], gsem.at[b])

    def gather_wait(b):
        pltpu.make_async_copy(tok_hbm.at[pl.ds(0, BB)], gbuf.at[b],
                              gsem.at[b]).wait()

    def scatter_start(u, b):
        pltpu.async_copy(tbuf.at[b], out_hbm.at[l0 + u, :, bhi], ssem.at[b])

    def scatter_wait(b):
        pltpu.make_async_copy(tbuf.at[b], out_hbm.at[0, :, 0], ssem.at[b]).wait()

    lane_ids = jnp.arange(VECL, dtype=jnp.int32)
    dhis = [(16 * j + lane_ids) >> 3 for j in range(D // VECL)]
    dlos = [(16 * j + lane_ids) & 7 for j in range(D // VECL)]

    def transpose_add(u, b):
        pvecs = [pos_v[pl.ds(u * D + 16 * j, VECL)] for j in range(D // VECL)]

        @plsc.parallel_loop(0, BB, unroll=4)
        def _(bl):
            blv = jnp.full((VECL,), bl, jnp.int32)
            for j in range(D // VECL):
                v = gbuf[b, bl, pl.ds(16 * j, VECL)] + pvecs[j]
                plsc.store_scatter(tbuf.at[b], [dhis[j], dlos[j], blv], v)

    gather_start(0, 0)

    @pl.loop(0, L_PER_W, step=2)
    def _(uu):
        for b in range(2):
            u = uu + b

            gather_wait(b)

            @pl.when(u + 1 < L_PER_W)
            def _():
                gather_start(u + 1, 1 - b)

            @pl.when(u >= 2)
            def _():
                scatter_wait(b)

            transpose_add(u, b)
            scatter_start(u, b)

    scatter_wait(0)
    scatter_wait(1)


def kernel(seq, token_table, pos_table):
    out5 = _seq_embed(seq.T, token_table, pos_table.reshape(-1))
    return jnp.transpose(out5, (2, 4, 0, 1, 3)).reshape(B, L, D)


# speedup vs baseline: 1.2557x; 1.2557x over previous
import functools

import jax
import jax.numpy as jnp
from jax import lax
from jax.experimental import pallas as pl
from jax.experimental.pallas import tpu as pltpu
from jax.experimental.pallas import tpu_sc as plsc

NC, NS = 2, 16
NW = NC * NS
B, L, D = 1024, 200, 64
VECL = 16
BB = 128
NB = B // BB
L_PER_W = L // (NW // NB)

_mesh = plsc.VectorSubcoreMesh(core_axis_name="c", subcore_axis_name="s")


@functools.partial(
    pl.kernel,
    out_type=jax.ShapeDtypeStruct((L, D // 8, NB, 8, BB), jnp.float32),
    mesh=_mesh,
    compiler_params=pltpu.CompilerParams(use_tc_tiling_on_sc=False,
                                        needs_layout_passes=False),
    scratch_types=[
        pltpu.VMEM((L_PER_W, BB), jnp.int32),
        pltpu.VMEM((L_PER_W * D,), jnp.float32),
        pltpu.VMEM((2, BB, D), jnp.float32),
        pltpu.VMEM((2, D // 8, 8, BB), jnp.float32),
        pltpu.SemaphoreType.DMA((2,)),
        pltpu.SemaphoreType.DMA((2,)),
    ],
)
def _seq_embed(seq_t_hbm, tok_hbm, pos_hbm, out_hbm, idx_v, pos_v, gbuf, tbuf,
               gsem, ssem):
    wid = lax.axis_index("s") * NC + lax.axis_index("c")
    l0 = (wid // NB) * L_PER_W
    bhi = wid % NB

    pltpu.sync_copy(seq_t_hbm.at[pl.ds(l0, L_PER_W), pl.ds(bhi * BB, BB)], idx_v)
    pltpu.sync_copy(pos_hbm.at[pl.ds(l0 * D, L_PER_W * D)], pos_v)

    def gather_start(u, b):
        pltpu.async_copy(tok_hbm.at[idx_v.at[u]], gbuf.at[b], gsem.at[b])

    def gather_wait(b):
        pltpu.make_async_copy(tok_hbm.at[pl.ds(0, BB)], gbuf.at[b],
                              gsem.at[b]).wait()

    def scatter_start(u, b):
        pltpu.async_copy(tbuf.at[b], out_hbm.at[l0 + u, :, bhi], ssem.at[b])

    def scatter_wait(b):
        pltpu.make_async_copy(tbuf.at[b], out_hbm.at[0, :, 0], ssem.at[b]).wait()

    lane_ids = jnp.arange(VECL, dtype=jnp.int32)
    blvecs = [lane_ids + 16 * g for g in range(BB // VECL)]

    def transpose_add(u, b):
        pvecs = [pos_v[pl.ds(u * D + 16 * j, VECL)] for j in range(D // VECL)]

        @pl.loop(0, VECL, init_carry=lane_ids, unroll=2)
        def _(s, perm):
            for j in range(D // VECL):
                dvec = perm + 16 * j
                dhi = dvec >> 3
                dlo = dvec & 7
                p = pvecs[j].at[perm].get(mode="promise_in_bounds")
                for g in range(BB // VECL):
                    v = plsc.load_gather(gbuf.at[b], [blvecs[g], dvec])
                    plsc.store_scatter(tbuf.at[b], [dhi, dlo, blvecs[g]], v + p)
            return (perm + 1) & 15

    gather_start(0, 0)

    @pl.loop(0, L_PER_W, step=2)
    def _(uu):
        for b in range(2):
            u = uu + b

            gather_wait(b)

            @pl.when(u + 1 < L_PER_W)
            def _():
                gather_start(u + 1, 1 - b)

            @pl.when(u >= 2)
            def _():
                scatter_wait(b)

            transpose_add(u, b)
            scatter_start(u, b)

    scatter_wait(0)
    scatter_wait(1)


def kernel(seq, token_table, pos_table):
    out5 = _seq_embed(seq.T, token_table, pos_table.reshape(-1))
    return jnp.transpose(out5, (2, 4, 0, 1, 3)).reshape(B, L, D)


# speedup vs baseline: 1.2565x; 1.0006x over previous
import functools

import jax
import jax.numpy as jnp
from jax import lax
from jax.experimental import pallas as pl
from jax.experimental.pallas import tpu as pltpu
from jax.experimental.pallas import tpu_sc as plsc

NC, NS = 2, 16
NW = NC * NS
B, L, D = 1024, 200, 64
VECL = 16
BB = 128
NB = B // BB
L_PER_W = L // (NW // NB)

_mesh = plsc.VectorSubcoreMesh(core_axis_name="c", subcore_axis_name="s")


@functools.partial(
    pl.kernel,
    out_type=jax.ShapeDtypeStruct((L, D // 8, NB, 8, BB), jnp.float32),
    mesh=_mesh,
    compiler_params=pltpu.CompilerParams(use_tc_tiling_on_sc=False,
                                         needs_layout_passes=False),
    scratch_types=[
        pltpu.VMEM((L_PER_W, BB), jnp.int32),
        pltpu.VMEM((L_PER_W * D,), jnp.float32),
        pltpu.VMEM((2, BB, D), jnp.float32),
        pltpu.VMEM((2, D, BB), jnp.float32),
        pltpu.SemaphoreType.DMA((2,)),
        pltpu.SemaphoreType.DMA((2,)),
    ],
)
def _seq_embed(seq_t_hbm, tok_hbm, pos_hbm, out_hbm, idx_v, pos_v, gbuf, tbuf,
               gsem, ssem):
    wid = lax.axis_index("s") * NC + lax.axis_index("c")
    l0 = (wid // NB) * L_PER_W
    bhi = wid % NB

    pltpu.sync_copy(seq_t_hbm.at[pl.ds(l0, L_PER_W), pl.ds(bhi * BB, BB)], idx_v)
    pltpu.sync_copy(pos_hbm.at[pl.ds(l0 * D, L_PER_W * D)], pos_v)

    def gather_start(u, b):
        pltpu.async_copy(tok_hbm.at[idx_v.at[u]], gbuf.at[b], gsem.at[b])

    def gather_wait(b):
        pltpu.make_async_copy(tok_hbm.at[pl.ds(0, BB)], gbuf.at[b],
                              gsem.at[b]).wait()

    def scatter_start(u, b):
        for dhi in range(D // 8):
            pltpu.async_copy(tbuf.at[b, pl.ds(8 * dhi, 8)],
                             out_hbm.at[l0 + u, dhi, bhi], ssem.at[b])

    def scatter_wait(b):
        for dhi in range(D // 8):
            pltpu.make_async_copy(tbuf.at[b, pl.ds(8 * dhi, 8)],
                                  out_hbm.at[0, 0, 0], ssem.at[b]).wait()

    lane_ids = jnp.arange(VECL, dtype=jnp.int32)
    blvecs = [lane_ids + 16 * g for g in range(BB // VECL)]

    def transpose_add(u, b):
        pvecs = [pos_v[pl.ds(u * D + 16 * j, VECL)] for j in range(D // VECL)]

        @pl.loop(0, VECL, init_carry=lane_ids, unroll=4)
        def _(s, perm):
            for j in range(D // VECL):
                dvec = perm + 16 * j
                p = pvecs[j].at[perm].get(mode="promise_in_bounds")
                for g in range(BB // VECL):
                    v = plsc.load_gather(gbuf.at[b], [blvecs[g], dvec])
                    plsc.store_scatter(tbuf.at[b], [dvec, blvecs[g]], v + p)
            return (perm + 1) & 15

    gather_start(0, 0)

    @pl.loop(0, L_PER_W, step=2)
    def _(uu):
        for b in range(2):
            u = uu + b

            gather_wait(b)

            @pl.when(u + 1 < L_PER_W)
            def _():
                gather_start(u + 1, 1 - b)

            @pl.when(u >= 2)
            def _():
                scatter_wait(b)

            transpose_add(u, b)
            scatter_start(u, b)

    scatter_wait(0)
    scatter_wait(1)


def kernel(seq, token_table, pos_table):
    out5 = _seq_embed(seq.T, token_table, pos_table.reshape(-1))
    return jnp.transpose(out5, (2, 4, 0, 1, 3)).reshape(B, L, D)


# speedup vs baseline: 1.6714x; 1.3302x over previous
import functools

import jax
import jax.numpy as jnp
from jax import lax
from jax.experimental import pallas as pl
from jax.experimental.pallas import tpu as pltpu
from jax.experimental.pallas import tpu_sc as plsc

NC, NS = 2, 16
NW = NC * NS
B, L, D = 1024, 200, 64
VECL = 16
BB = 128
NB = B // BB
L_PER_W = L // (NW // NB)

_mesh = plsc.VectorSubcoreMesh(core_axis_name="c", subcore_axis_name="s")


@functools.partial(
    pl.kernel,
    out_type=jax.ShapeDtypeStruct((L, D // 8, NB, 8, BB), jnp.float32),
    mesh=_mesh,
    compiler_params=pltpu.CompilerParams(use_tc_tiling_on_sc=False,
                                         needs_layout_passes=False),
    scratch_types=[
        pltpu.VMEM((L_PER_W, BB), jnp.int32),
        pltpu.VMEM((L_PER_W * D,), jnp.float32),
        pltpu.VMEM((2, BB, D), jnp.float32),
        pltpu.VMEM((2, D, BB), jnp.float32),
        pltpu.SemaphoreType.DMA((2,)),
        pltpu.SemaphoreType.DMA((2,)),
    ],
)
def _seq_embed(seq_t_hbm, tok_hbm, pos_hbm, out_hbm, idx_v, pos_v, gbuf, tbuf,
               gsem, ssem):
    wid = lax.axis_index("s") * NC + lax.axis_index("c")
    l0 = (wid // NB) * L_PER_W
    bhi = wid % NB

    pltpu.sync_copy(seq_t_hbm.at[pl.ds(l0, L_PER_W), pl.ds(bhi * BB, BB)], idx_v)
    pltpu.sync_copy(pos_hbm.at[pl.ds(l0 * D, L_PER_W * D)], pos_v)

    def gather_start(u, b):
        pltpu.async_copy(tok_hbm.at[idx_v.at[u]], gbuf.at[b], gsem.at[b])

    def gather_wait(b):
        pltpu.make_async_copy(tok_hbm.at[pl.ds(0, BB)], gbuf.at[b],
                              gsem.at[b]).wait()

    def scatter_start(u, b):
        for dhi in range(D // 8):
            pltpu.async_copy(tbuf.at[b, pl.ds(8 * dhi, 8)],
                             out_hbm.at[l0 + u, dhi, bhi], ssem.at[b])

    def scatter_wait(b):
        for dhi in range(D // 8):
            pltpu.make_async_copy(tbuf.at[b, pl.ds(8 * dhi, 8)],
                                  out_hbm.at[0, 0, 0], ssem.at[b]).wait()

    lane_ids = jnp.arange(VECL, dtype=jnp.int32)
    blvecs = [lane_ids + 16 * g for g in range(BB // VECL)]

    def transpose_add(u, b):
        pvecs = [pos_v[pl.ds(u * D + 16 * j, VECL)] for j in range(D // VECL)]

        @plsc.parallel_loop(0, VECL, unroll=4)
        def _(s, /):
            perm = (lane_ids + s) & 15
            for j in range(D // VECL):
                dvec = perm + 16 * j
                p = pvecs[j].at[perm].get(mode="promise_in_bounds")
                for g in range(BB // VECL):
                    v = plsc.load_gather(gbuf.at[b], [blvecs[g], dvec])
                    plsc.store_scatter(tbuf.at[b], [dvec, blvecs[g]], v + p)

    gather_start(0, 0)

    @pl.loop(0, L_PER_W, step=2)
    def _(uu):
        for b in range(2):
            u = uu + b

            gather_wait(b)

            @pl.when(u + 1 < L_PER_W)
            def _():
                gather_start(u + 1, 1 - b)

            @pl.when(u >= 2)
            def _():
                scatter_wait(b)

            transpose_add(u, b)
            scatter_start(u, b)

    scatter_wait(0)
    scatter_wait(1)


def kernel(seq, token_table, pos_table):
    out5 = _seq_embed(seq.T, token_table, pos_table.reshape(-1))
    return jnp.transpose(out5, (2, 4, 0, 1, 3)).reshape(B, L, D)


# speedup vs baseline: 1.8155x; 1.0863x over previous
import functools

import jax
import jax.numpy as jnp
from jax import lax
from jax.experimental import pallas as pl
from jax.experimental.pallas import tpu as pltpu
from jax.experimental.pallas import tpu_sc as plsc

NC, NS = 2, 16
NW = NC * NS
B, L, D = 1024, 200, 64
VECL = 16
BB = 128
NB = B // BB
L_PER_W = L // (NW // NB)

_mesh = plsc.VectorSubcoreMesh(core_axis_name="c", subcore_axis_name="s")


@functools.partial(
    pl.kernel,
    out_type=jax.ShapeDtypeStruct((L, D // 8, NB, 8, BB), jnp.float32),
    mesh=_mesh,
    compiler_params=pltpu.CompilerParams(use_tc_tiling_on_sc=False,
                                         needs_layout_passes=False),
    scratch_types=[
        pltpu.VMEM((L_PER_W, BB), jnp.int32),
        pltpu.VMEM((L_PER_W * D,), jnp.float32),
        pltpu.VMEM((2, BB, D), jnp.float32),
        pltpu.VMEM((2, D, BB), jnp.float32),
        pltpu.SemaphoreType.DMA((2,)),
        pltpu.SemaphoreType.DMA((2,)),
    ],
)
def _seq_embed(seq_t_hbm, tok_hbm, pos_hbm, out_hbm, idx_v, pos_v, gbuf, tbuf,
               gsem, ssem):
    wid = lax.axis_index("s") * NC + lax.axis_index("c")
    l0 = (wid // NB) * L_PER_W
    bhi = wid % NB

    pltpu.sync_copy(seq_t_hbm.at[pl.ds(l0, L_PER_W), pl.ds(bhi * BB, BB)], idx_v)
    pltpu.sync_copy(pos_hbm.at[pl.ds(l0 * D, L_PER_W * D)], pos_v)

    def gather_start(u, b):
        pltpu.async_copy(tok_hbm.at[idx_v.at[u]], gbuf.at[b], gsem.at[b])

    def gather_wait(b):
        pltpu.make_async_copy(tok_hbm.at[pl.ds(0, BB)], gbuf.at[b],
                              gsem.at[b]).wait()

    def scatter_start(u, b):
        for dhi in range(D // 8):
            pltpu.async_copy(tbuf.at[b, pl.ds(8 * dhi, 8)],
                             out_hbm.at[l0 + u, dhi, bhi], ssem.at[b])

    def scatter_wait(b):
        for dhi in range(D // 8):
            pltpu.make_async_copy(tbuf.at[b, pl.ds(8 * dhi, 8)],
                                  out_hbm.at[0, 0, 0], ssem.at[b]).wait()

    lane_ids = jnp.arange(VECL, dtype=jnp.int32)
    blvecs = [lane_ids + 16 * g for g in range(BB // VECL)]

    def transpose_add(u, b):
        pvecs = [pos_v[pl.ds(u * D + 16 * j, VECL)] for j in range(D // VECL)]

        @plsc.parallel_loop(0, VECL, unroll=8)
        def _(s, /):
            perm = (lane_ids + s) & 15
            for j in range(D // VECL):
                dvec = perm + 16 * j
                p = pvecs[j].at[perm].get(mode="promise_in_bounds")
                for g in range(BB // VECL):
                    v = plsc.load_gather(gbuf.at[b], [blvecs[g], dvec])
                    plsc.store_scatter(tbuf.at[b], [dvec, blvecs[g]], v + p)

    gather_start(0, 0)

    @pl.loop(0, L_PER_W, step=2)
    def _(uu):
        for b in range(2):
            u = uu + b

            gather_wait(b)

            @pl.when(u + 1 < L_PER_W)
            def _():
                gather_start(u + 1, 1 - b)

            @pl.when(u >= 2)
            def _():
                scatter_wait(b)

            transpose_add(u, b)
            scatter_start(u, b)

    scatter_wait(0)
    scatter_wait(1)


def kernel(seq, token_table, pos_table):
    out5 = _seq_embed(seq.T, token_table, pos_table.reshape(-1))
    return jnp.transpose(out5, (2, 4, 0, 1, 3)).reshape(B, L, D)
